# Initial kernel scaffold; baseline (speedup 1.0000x reference)
import jax
import jax.numpy as jnp
from jax.experimental import pallas as pl


def kernel(last_product_id, last_product_business_desc, last_product_type_desc, last_product_sub_category, last_product_list_price, last_image_embedding_pca, item_table, business_table, type_table, subcat_table, price_table, price_boundaries, cross_W, cross_b, dense_W, dense_b):
    raise NotImplementedError("write your pallas kernel here")


# speedup vs baseline: 1.7310x; 1.7310x over previous
import functools

import jax
import jax.numpy as jnp
from jax import lax
from jax.experimental import pallas as pl
from jax.experimental.pallas import tpu as pltpu
from jax.experimental.pallas import tpu_sc as plsc

B = 16384
EMB = 64
NC = 2
NS = 16
NW = NC * NS
BPW = B // NW
CH = 128
NCH = BPW // CH
IR = B // CH
NBND = 1024


def _sc_body(item_i, bus_i, typ_i, sub_i, price_h, bnd_h,
             item_t, bus_t, typ_t, sub_t, price_t,
             item_o, bus_o, typ_o, sub_o, price_o,
             idx_v, price_v, bnd_v, buf_a, buf_b, sem_a, sem_b):
    wid = lax.axis_index("s") * NC + lax.axis_index("c")
    rbase = wid * NCH
    obase = wid * BPW

    pltpu.sync_copy(item_i.at[pl.ds(rbase, NCH)], idx_v.at[pl.ds(0, NCH)])
    pltpu.sync_copy(bus_i.at[pl.ds(rbase, NCH)], idx_v.at[pl.ds(NCH, NCH)])
    pltpu.sync_copy(typ_i.at[pl.ds(rbase, NCH)], idx_v.at[pl.ds(2 * NCH, NCH)])
    pltpu.sync_copy(sub_i.at[pl.ds(rbase, NCH)], idx_v.at[pl.ds(3 * NCH, NCH)])
    pltpu.sync_copy(price_h.at[pl.ds(rbase, NCH)], price_v)
    pltpu.sync_copy(bnd_h, bnd_v)

    tabs = [item_t, bus_t, typ_t, sub_t, price_t]
    outs = [item_o, bus_o, typ_o, sub_o, price_o]
    bufs = [buf_a, buf_b]
    sems = [sem_a, sem_b]
    copies = [None, None]
    n_units = 5 * NCH

    def fire(k):
        t, j = divmod(k, NCH)
        copies[k % 2] = pltpu.async_copy(
            tabs[t].at[idx_v.at[t * NCH + j]], bufs[k % 2], sems[k % 2])

    fire(0)
    fire(1)

    for r in range(NCH):
        for c in range(CH // 16):
            v = price_v[r, pl.ds(c * 16, 16)]
            base = jnp.zeros((16,), jnp.int32)
            n = NBND
            while n > 1:
                half = n // 2
                probe = plsc.load_gather(bnd_v, [base + (half - 1)])
                base = base + jnp.where(probe < v, half, 0)
                n -= half
            probe = plsc.load_gather(bnd_v, [base])
            base = base + jnp.where(probe < v, 1, 0)
            idx_v[4 * NCH + r, pl.ds(c * 16, 16)] = base

    for k in range(n_units):
        copies[k % 2].wait()
        t, j = divmod(k, NCH)
        pltpu.sync_copy(bufs[k % 2], outs[t].at[pl.ds(obase + j * CH, CH)])
        if k + 2 < n_units:
            fire(k + 2)


def _sc_gather(item_i, bus_i, typ_i, sub_i, price_i, bnd,
               item_t, bus_t, typ_t, sub_t, price_t):
    row = jax.ShapeDtypeStruct((B, EMB), jnp.float32)
    f = functools.partial(
        pl.kernel,
        out_type=[row] * 5,
        mesh=plsc.VectorSubcoreMesh(core_axis_name="c", subcore_axis_name="s"),
        scratch_types=[
            pltpu.VMEM((5 * NCH, CH), jnp.int32),
            pltpu.VMEM((NCH, CH), jnp.float32),
            pltpu.VMEM((NBND,), jnp.float32),
            pltpu.VMEM((CH, EMB), jnp.float32),
            pltpu.VMEM((CH, EMB), jnp.float32),
            pltpu.SemaphoreType.DMA,
            pltpu.SemaphoreType.DMA,
        ],
        compiler_params=pltpu.CompilerParams(needs_layout_passes=False,
                                             use_tc_tiling_on_sc=False),
        name="item_model_sc_gather",
    )(_sc_body)
    return f(item_i, bus_i, typ_i, sub_i, price_i, bnd,
             item_t, bus_t, typ_t, sub_t, price_t)


def _tc_body(item_r, bus_r, typ_r, sub_r, price_r, img_r,
             wc_r, bc_r, wd_r, bd_r, out_r):
    attrs = jnp.concatenate([bus_r[...], typ_r[...], sub_r[...]], axis=1)
    u = jnp.dot(attrs, wc_r[...], preferred_element_type=jnp.float32) + bc_r[...]
    cross = attrs * u + attrs
    img = jnp.dot(img_r[...], wd_r[...], preferred_element_type=jnp.float32)
    img = jnp.maximum(img + bd_r[...], 0.0)
    out_r[...] = jnp.concatenate([item_r[...], cross, price_r[...], img], axis=1)


def _tc_combine(item_r, bus_r, typ_r, sub_r, price_r, img,
                cross_W, cross_b, dense_W, dense_b):
    blk = 1024
    grid = B // blk
    rows = pl.BlockSpec((blk, EMB), lambda i: (i, 0))
    return pl.pallas_call(
        _tc_body,
        grid=(grid,),
        in_specs=[
            rows, rows, rows, rows, rows,
            pl.BlockSpec((blk, 12), lambda i: (i, 0)),
            pl.BlockSpec((3 * EMB, 3 * EMB), lambda i: (0, 0)),
            pl.BlockSpec((1, 3 * EMB), lambda i: (0, 0)),
            pl.BlockSpec((12, 12), lambda i: (0, 0)),
            pl.BlockSpec((1, 12), lambda i: (0, 0)),
        ],
        out_specs=pl.BlockSpec((blk, 332), lambda i: (i, 0)),
        out_shape=jax.ShapeDtypeStruct((B, 332), jnp.float32),
    )(item_r, bus_r, typ_r, sub_r, price_r, img,
      cross_W, cross_b, dense_W, dense_b)


def kernel(last_product_id, last_product_business_desc, last_product_type_desc,
           last_product_sub_category, last_product_list_price,
           last_image_embedding_pca, item_table, business_table, type_table,
           subcat_table, price_table, price_boundaries, cross_W, cross_b,
           dense_W, dense_b):
    item_i = last_product_id.reshape(IR, CH)
    bus_i = last_product_business_desc.reshape(IR, CH)
    typ_i = last_product_type_desc.reshape(IR, CH)
    sub_i = last_product_sub_category.reshape(IR, CH)
    price_i = last_product_list_price.reshape(IR, CH)
    bnd = jnp.concatenate(
        [price_boundaries,
         jnp.full((NBND - price_boundaries.shape[0],), jnp.inf, jnp.float32)])
    item_r, bus_r, typ_r, sub_r, price_r = _sc_gather(
        item_i, bus_i, typ_i, sub_i, price_i, bnd,
        item_table, business_table, type_table, subcat_table, price_table)
    return _tc_combine(item_r, bus_r, typ_r, sub_r, price_r,
                       last_image_embedding_pca, cross_W,
                       cross_b.reshape(1, 3 * EMB), dense_W,
                       dense_b.reshape(1, 12))


# speedup vs baseline: 1.7623x; 1.0181x over previous
import functools

import jax
import jax.numpy as jnp
from jax import lax
from jax.experimental import pallas as pl
from jax.experimental.pallas import tpu as pltpu
from jax.experimental.pallas import tpu_sc as plsc

B = 16384
EMB = 64
ITEM_V = 1000000
NC = 2
NS = 16
NW = NC * NS
BPW = B // NW
CH = 128
NCH = BPW // CH
IR = B // CH
NBND = 1024


def _sc_item_body(item_i, item_t, item_o, idx_v, pair_a, pair_b, sem_a, sem_b):
    wid = lax.axis_index("s") * NC + lax.axis_index("c")
    rbase = wid * NCH
    obase = wid * BPW
    pltpu.sync_copy(item_i.at[pl.ds(rbase, NCH)], idx_v)
    bufs = [pair_a, pair_b]
    sems = [sem_a, sem_b]
    copies = [None, None]

    def fire(j):
        copies[j % 2] = pltpu.async_copy(
            item_t.at[idx_v.at[j]], bufs[j % 2], sems[j % 2])

    fire(0)
    fire(1)
    for j in range(NCH):
        copies[j % 2].wait()
        pltpu.sync_copy(bufs[j % 2], item_o.at[pl.ds(obase + j * CH, CH)])
        if j + 2 < NCH:
            fire(j + 2)


def _sc_body(bus_i, typ_i, sub_i, price_h, bnd_h,
             bus_t, typ_t, sub_t, price_t,
             bus_o, typ_o, sub_o, price_o,
             idx_v, price_v, bnd_v, buf_a, buf_b, sem_a, sem_b):
    wid = lax.axis_index("s") * NC + lax.axis_index("c")
    rbase = wid * NCH
    obase = wid * BPW

    pltpu.sync_copy(bus_i.at[pl.ds(rbase, NCH)], idx_v.at[pl.ds(0, NCH)])
    pltpu.sync_copy(typ_i.at[pl.ds(rbase, NCH)], idx_v.at[pl.ds(NCH, NCH)])
    pltpu.sync_copy(sub_i.at[pl.ds(rbase, NCH)], idx_v.at[pl.ds(2 * NCH, NCH)])
    pltpu.sync_copy(price_h.at[pl.ds(rbase, NCH)], price_v)
    pltpu.sync_copy(bnd_h, bnd_v)

    tabs = [bus_t, typ_t, sub_t, price_t]
    outs = [bus_o, typ_o, sub_o, price_o]
    bufs = [buf_a, buf_b]
    sems = [sem_a, sem_b]
    copies = [None, None]
    n_units = 4 * NCH

    def fire(k):
        t, j = divmod(k, NCH)
        copies[k % 2] = pltpu.async_copy(
            tabs[t].at[idx_v.at[t * NCH + j]], bufs[k % 2], sems[k % 2])

    fire(0)
    fire(1)

    for r in range(NCH):
        for c in range(CH // 16):
            v = price_v[r, pl.ds(c * 16, 16)]
            base = jnp.zeros((16,), jnp.int32)
            n = NBND
            while n > 1:
                half = n // 2
                probe = plsc.load_gather(bnd_v, [base + (half - 1)])
                base = base + jnp.where(probe < v, half, 0)
                n -= half
            probe = plsc.load_gather(bnd_v, [base])
            base = base + jnp.where(probe < v, 1, 0)
            idx_v[3 * NCH + r, pl.ds(c * 16, 16)] = base

    for k in range(n_units):
        copies[k % 2].wait()
        t, j = divmod(k, NCH)
        pltpu.sync_copy(bufs[k % 2], outs[t].at[pl.ds(obase + j * CH, CH)])
        if k + 2 < n_units:
            fire(k + 2)


def _sc_item_gather(item_i, item_t):
    f = functools.partial(
        pl.kernel,
        out_type=jax.ShapeDtypeStruct((B, 2 * EMB), jnp.float32),
        mesh=plsc.VectorSubcoreMesh(core_axis_name="c", subcore_axis_name="s"),
        scratch_types=[
            pltpu.VMEM((NCH, CH), jnp.int32),
            pltpu.VMEM((CH, 2 * EMB), jnp.float32),
            pltpu.VMEM((CH, 2 * EMB), jnp.float32),
            pltpu.SemaphoreType.DMA,
            pltpu.SemaphoreType.DMA,
        ],
        compiler_params=pltpu.CompilerParams(needs_layout_passes=False,
                                             use_tc_tiling_on_sc=True),
        name="item_model_sc_item_gather",
    )(_sc_item_body)
    return f(item_i, item_t)


def _sc_gather(bus_i, typ_i, sub_i, price_i, bnd,
               bus_t, typ_t, sub_t, price_t):
    row = jax.ShapeDtypeStruct((B, EMB), jnp.float32)
    f = functools.partial(
        pl.kernel,
        out_type=[row] * 4,
        mesh=plsc.VectorSubcoreMesh(core_axis_name="c", subcore_axis_name="s"),
        scratch_types=[
            pltpu.VMEM((4 * NCH, CH), jnp.int32),
            pltpu.VMEM((NCH, CH), jnp.float32),
            pltpu.VMEM((NBND,), jnp.float32),
            pltpu.VMEM((CH, EMB), jnp.float32),
            pltpu.VMEM((CH, EMB), jnp.float32),
            pltpu.SemaphoreType.DMA,
            pltpu.SemaphoreType.DMA,
        ],
        compiler_params=pltpu.CompilerParams(needs_layout_passes=False,
                                             use_tc_tiling_on_sc=False),
        name="item_model_sc_gather",
    )(_sc_body)
    return f(bus_i, typ_i, sub_i, price_i, bnd, bus_t, typ_t, sub_t, price_t)


def _tc_body(pair_r, par_r, bus_r, typ_r, sub_r, price_r, img_r,
             wc_r, bc_r, wd_r, bd_r, out_r):
    p = par_r[...]
    item = pair_r[:, 0:EMB] * (1.0 - p) + pair_r[:, EMB:2 * EMB] * p
    attrs = jnp.concatenate([bus_r[...], typ_r[...], sub_r[...]], axis=1)
    u = jnp.dot(attrs, wc_r[...], preferred_element_type=jnp.float32) + bc_r[...]
    cross = attrs * u + attrs
    img = jnp.dot(img_r[...], wd_r[...], preferred_element_type=jnp.float32)
    img = jnp.maximum(img + bd_r[...], 0.0)
    out_r[...] = jnp.concatenate([item, cross, price_r[...], img], axis=1)


def _tc_combine(pair_r, par, bus_r, typ_r, sub_r, price_r, img,
                cross_W, cross_b, dense_W, dense_b):
    blk = 1024
    grid = B // blk
    rows = pl.BlockSpec((blk, EMB), lambda i: (i, 0))
    return pl.pallas_call(
        _tc_body,
        grid=(grid,),
        in_specs=[
            pl.BlockSpec((blk, 2 * EMB), lambda i: (i, 0)),
            pl.BlockSpec((blk, 1), lambda i: (i, 0)),
            rows, rows, rows, rows,
            pl.BlockSpec((blk, 12), lambda i: (i, 0)),
            pl.BlockSpec((3 * EMB, 3 * EMB), lambda i: (0, 0)),
            pl.BlockSpec((1, 3 * EMB), lambda i: (0, 0)),
            pl.BlockSpec((12, 12), lambda i: (0, 0)),
            pl.BlockSpec((1, 12), lambda i: (0, 0)),
        ],
        out_specs=pl.BlockSpec((blk, 332), lambda i: (i, 0)),
        out_shape=jax.ShapeDtypeStruct((B, 332), jnp.float32),
    )(pair_r, par, bus_r, typ_r, sub_r, price_r, img,
      cross_W, cross_b, dense_W, dense_b)


def kernel(last_product_id, last_product_business_desc, last_product_type_desc,
           last_product_sub_category, last_product_list_price,
           last_image_embedding_pca, item_table, business_table, type_table,
           subcat_table, price_table, price_boundaries, cross_W, cross_b,
           dense_W, dense_b):
    item_i = (last_product_id >> 1).reshape(IR, CH)
    bus_i = last_product_business_desc.reshape(IR, CH)
    typ_i = last_product_type_desc.reshape(IR, CH)
    sub_i = last_product_sub_category.reshape(IR, CH)
    price_i = last_product_list_price.reshape(IR, CH)
    bnd = jnp.concatenate(
        [price_boundaries,
         jnp.full((NBND - price_boundaries.shape[0],), jnp.inf, jnp.float32)])
    item_pairs_t = item_table.reshape(ITEM_V // 2, 2 * EMB)
    par = (last_product_id & 1).astype(jnp.float32).reshape(B, 1)
    pair_r = _sc_item_gather(item_i, item_pairs_t)
    bus_r, typ_r, sub_r, price_r = _sc_gather(
        bus_i, typ_i, sub_i, price_i, bnd,
        business_table, type_table, subcat_table, price_table)
    return _tc_combine(pair_r, par, bus_r, typ_r, sub_r, price_r,
                       last_image_embedding_pca, cross_W,
                       cross_b.reshape(1, 3 * EMB), dense_W,
                       dense_b.reshape(1, 12))
